# Initial kernel scaffold; baseline (speedup 1.0000x reference)
import jax
import jax.numpy as jnp
from jax.experimental import pallas as pl


def kernel(tokens, table, W, b):
    raise NotImplementedError("write your pallas kernel here")


# speedup vs baseline: 5.4342x; 5.4342x over previous
import functools
import math

import jax
import jax.numpy as jnp
from jax import lax
from jax.experimental import pallas as pl
from jax.experimental.pallas import tpu as pltpu
from jax.experimental.pallas import tpu_sc as plsc

_VOCAB = 100000
_EMB = 300
_OUT = 128
_SCALE = math.sqrt(300.0)

_ROW_BLK = 2000


def _proj_body(t_ref, w_ref, b_ref, o_ref):
    acc = jnp.dot(t_ref[...], w_ref[...], preferred_element_type=jnp.float32)
    o_ref[...] = acc * _SCALE + b_ref[...]


def _project_table(table, W, b):
    grid = _VOCAB // _ROW_BLK
    return pl.pallas_call(
        _proj_body,
        grid=(grid,),
        in_specs=[
            pl.BlockSpec((_ROW_BLK, _EMB), lambda i: (i, 0)),
            pl.BlockSpec((_EMB, _OUT), lambda i: (0, 0)),
            pl.BlockSpec((1, _OUT), lambda i: (0, 0)),
        ],
        out_specs=pl.BlockSpec((_ROW_BLK, _OUT), lambda i: (i, 0)),
        out_shape=jax.ShapeDtypeStruct((_VOCAB, _OUT), jnp.float32),
    )(table, W, b.reshape(1, _OUT))


_NTOK = 4096 * 200
_NW = 32
_PER_W = _NTOK // _NW
_CHUNK = 400
_NCHUNK = _PER_W // _CHUNK


def _gather_kernel(tok_hbm, p_hbm, out_hbm, idx_v, rows_v, sem):
    wid = lax.axis_index("s") * 2 + lax.axis_index("c")
    base = wid * _PER_W

    def step(g, _):
        off = base + g * _CHUNK
        pltpu.sync_copy(tok_hbm.at[pl.ds(off, _CHUNK)], idx_v)
        pltpu.async_copy(p_hbm.at[idx_v], rows_v, sem).wait()
        pltpu.sync_copy(rows_v, out_hbm.at[pl.ds(off, _CHUNK)])
        return 0

    lax.fori_loop(0, _NCHUNK, step, 0)


@functools.partial(jax.jit, static_argnames=())
def kernel(tokens, table, W, b):
    proj = _project_table(table, W, b)
    tok_flat = tokens.reshape(_NTOK)

    sc_gather = pl.kernel(
        _gather_kernel,
        out_type=jax.ShapeDtypeStruct((_NTOK, _OUT), jnp.float32),
        mesh=plsc.VectorSubcoreMesh(core_axis_name="c", subcore_axis_name="s"),
        scratch_types=[
            pltpu.VMEM((_CHUNK,), jnp.int32),
            pltpu.VMEM((_CHUNK, _OUT), jnp.float32),
            pltpu.SemaphoreType.DMA,
        ],
    )
    out_flat = sc_gather(tok_flat, proj)
    return out_flat.reshape(4096, 200, _OUT)


# speedup vs baseline: 6.0685x; 1.1167x over previous
import functools
import math

import jax
import jax.numpy as jnp
from jax import lax
from jax.experimental import pallas as pl
from jax.experimental.pallas import tpu as pltpu
from jax.experimental.pallas import tpu_sc as plsc

_VOCAB = 100000
_EMB = 300
_OUT = 128
_SCALE = math.sqrt(300.0)

_ROW_BLK = 2000


def _proj_body(t_ref, w_ref, b_ref, o_ref):
    acc = jnp.dot(t_ref[...], w_ref[...], preferred_element_type=jnp.float32)
    o_ref[...] = acc * _SCALE + b_ref[...]


def _project_table(table, W, b):
    grid = _VOCAB // _ROW_BLK
    return pl.pallas_call(
        _proj_body,
        grid=(grid,),
        in_specs=[
            pl.BlockSpec((_ROW_BLK, _EMB), lambda i: (i, 0)),
            pl.BlockSpec((_EMB, _OUT), lambda i: (0, 0)),
            pl.BlockSpec((1, _OUT), lambda i: (0, 0)),
        ],
        out_specs=pl.BlockSpec((_ROW_BLK, _OUT), lambda i: (i, 0)),
        out_shape=jax.ShapeDtypeStruct((_VOCAB, _OUT), jnp.float32),
    )(table, W, b.reshape(1, _OUT))


_NTOK = 4096 * 200
_NW = 32
_PER_W = _NTOK // _NW
_CHUNK = 400
_NCHUNK = _PER_W // _CHUNK


def _gather_kernel(tok_hbm, p_hbm, out_hbm, idx0, idx1, rows0, rows1, sem0, sem1):
    wid = lax.axis_index("s") * 2 + lax.axis_index("c")
    base = wid * _PER_W
    idx = (idx0, idx1)
    rows = (rows0, rows1)
    sem = (sem0, sem1)

    for b in range(2):
        pltpu.sync_copy(tok_hbm.at[pl.ds(base + b * _CHUNK, _CHUNK)], idx[b])
        pltpu.async_copy(p_hbm.at[idx[b]], rows[b], sem[b])

    @pl.loop(0, _NCHUNK - 2, step=2)
    def main(g):
        for b in range(2):
            c = g + b
            pltpu.make_async_copy(p_hbm.at[idx[b]], rows[b], sem[b]).wait()
            pltpu.sync_copy(rows[b], out_hbm.at[pl.ds(base + c * _CHUNK, _CHUNK)])
            pltpu.sync_copy(
                tok_hbm.at[pl.ds(base + (c + 2) * _CHUNK, _CHUNK)], idx[b]
            )
            pltpu.async_copy(p_hbm.at[idx[b]], rows[b], sem[b])

    for b in range(2):
        c = _NCHUNK - 2 + b
        pltpu.make_async_copy(p_hbm.at[idx[b]], rows[b], sem[b]).wait()
        pltpu.sync_copy(rows[b], out_hbm.at[pl.ds(base + c * _CHUNK, _CHUNK)])


@functools.partial(jax.jit, static_argnames=())
def kernel(tokens, table, W, b):
    proj = _project_table(table, W, b)
    tok_flat = tokens.reshape(_NTOK)

    sc_gather = pl.kernel(
        _gather_kernel,
        out_type=jax.ShapeDtypeStruct((_NTOK, _OUT), jnp.float32),
        mesh=plsc.VectorSubcoreMesh(core_axis_name="c", subcore_axis_name="s"),
        scratch_types=[
            pltpu.VMEM((_CHUNK,), jnp.int32),
            pltpu.VMEM((_CHUNK,), jnp.int32),
            pltpu.VMEM((_CHUNK, _OUT), jnp.float32),
            pltpu.VMEM((_CHUNK, _OUT), jnp.float32),
            pltpu.SemaphoreType.DMA,
            pltpu.SemaphoreType.DMA,
        ],
    )
    out_flat = sc_gather(tok_flat, proj)
    return out_flat.reshape(4096, 200, _OUT)


# speedup vs baseline: 6.2170x; 1.0245x over previous
import functools
import math

import jax
import jax.numpy as jnp
from jax import lax
from jax.experimental import pallas as pl
from jax.experimental.pallas import tpu as pltpu
from jax.experimental.pallas import tpu_sc as plsc

_VOCAB = 100000
_EMB = 300
_OUT = 128
_SCALE = math.sqrt(300.0)

_ROW_BLK = 5000


def _proj_body(t_ref, w_ref, b_ref, o_ref):
    acc = jnp.dot(t_ref[...], w_ref[...], preferred_element_type=jnp.float32)
    o_ref[...] = acc * _SCALE + b_ref[...]


def _project_table(table, W, b):
    grid = _VOCAB // _ROW_BLK
    return pl.pallas_call(
        _proj_body,
        grid=(grid,),
        in_specs=[
            pl.BlockSpec((_ROW_BLK, _EMB), lambda i: (i, 0)),
            pl.BlockSpec((_EMB, _OUT), lambda i: (0, 0)),
            pl.BlockSpec((1, _OUT), lambda i: (0, 0)),
        ],
        out_specs=pl.BlockSpec((_ROW_BLK, _OUT), lambda i: (i, 0)),
        out_shape=jax.ShapeDtypeStruct((_VOCAB, _OUT), jnp.float32),
    )(table, W, b.reshape(1, _OUT))


_NTOK = 4096 * 200
_NW = 32
_PER_W = _NTOK // _NW
_CHUNK = 400
_NCHUNK = _PER_W // _CHUNK


def _gather_kernel(tok_hbm, p_hbm, out_hbm, idx0, idx1, rows0, rows1, sem0, sem1):
    wid = lax.axis_index("s") * 2 + lax.axis_index("c")
    base = wid * _PER_W
    idx = (idx0, idx1)
    rows = (rows0, rows1)
    sem = (sem0, sem1)

    for b in range(2):
        pltpu.sync_copy(tok_hbm.at[pl.ds(base + b * _CHUNK, _CHUNK)], idx[b])
        pltpu.async_copy(p_hbm.at[idx[b]], rows[b], sem[b])

    @pl.loop(0, _NCHUNK - 2, step=2)
    def main(g):
        for b in range(2):
            c = g + b
            pltpu.make_async_copy(p_hbm.at[idx[b]], rows[b], sem[b]).wait()
            pltpu.sync_copy(rows[b], out_hbm.at[pl.ds(base + c * _CHUNK, _CHUNK)])
            pltpu.sync_copy(
                tok_hbm.at[pl.ds(base + (c + 2) * _CHUNK, _CHUNK)], idx[b]
            )
            pltpu.async_copy(p_hbm.at[idx[b]], rows[b], sem[b])

    for b in range(2):
        c = _NCHUNK - 2 + b
        pltpu.make_async_copy(p_hbm.at[idx[b]], rows[b], sem[b]).wait()
        pltpu.sync_copy(rows[b], out_hbm.at[pl.ds(base + c * _CHUNK, _CHUNK)])


@functools.partial(jax.jit, static_argnames=())
def kernel(tokens, table, W, b):
    proj = _project_table(table, W, b)
    tok_flat = tokens.reshape(_NTOK)

    sc_gather = pl.kernel(
        _gather_kernel,
        out_type=jax.ShapeDtypeStruct((_NTOK, _OUT), jnp.float32),
        mesh=plsc.VectorSubcoreMesh(core_axis_name="c", subcore_axis_name="s"),
        scratch_types=[
            pltpu.VMEM((_CHUNK,), jnp.int32),
            pltpu.VMEM((_CHUNK,), jnp.int32),
            pltpu.VMEM((_CHUNK, _OUT), jnp.float32),
            pltpu.VMEM((_CHUNK, _OUT), jnp.float32),
            pltpu.SemaphoreType.DMA,
            pltpu.SemaphoreType.DMA,
        ],
    )
    out_flat = sc_gather(tok_flat, proj)
    return out_flat.reshape(4096, 200, _OUT)


# speedup vs baseline: 6.2412x; 1.0039x over previous
import functools
import math

import jax
import jax.numpy as jnp
from jax import lax
from jax.experimental import pallas as pl
from jax.experimental.pallas import tpu as pltpu
from jax.experimental.pallas import tpu_sc as plsc

_VOCAB = 100000
_EMB = 300
_OUT = 128
_SCALE = math.sqrt(300.0)

_PROJ_ROW = 2000
_PROJ_NB = _VOCAB // _PROJ_ROW
_RING = 5


def _proj_body(t_any, w_v, b_v, o_any, *scr):
    tbuf = scr[0:_RING]
    obuf = scr[_RING:2 * _RING]
    tsem = scr[2 * _RING:3 * _RING]
    osem = scr[3 * _RING:4 * _RING]

    for s in range(_RING):
        pltpu.async_copy(
            t_any.at[pl.ds(s * _PROJ_ROW, _PROJ_ROW)], tbuf[s], tsem[s]
        )

    @pl.loop(0, _PROJ_NB, step=_RING)
    def main(g):
        for s in range(_RING):
            c = g + s
            pltpu.make_async_copy(
                t_any.at[pl.ds(0, _PROJ_ROW)], tbuf[s], tsem[s]
            ).wait()

            @pl.when(c >= _RING)
            def _wait_prev_write():
                pltpu.make_async_copy(
                    obuf[s], o_any.at[pl.ds(0, _PROJ_ROW)], osem[s]
                ).wait()

            acc = jnp.dot(tbuf[s][...], w_v[...], preferred_element_type=jnp.float32)
            obuf[s][...] = acc * _SCALE + b_v[...]
            pltpu.async_copy(
                obuf[s], o_any.at[pl.ds(c * _PROJ_ROW, _PROJ_ROW)], osem[s]
            )

            @pl.when(c + _RING < _PROJ_NB)
            def _refill():
                pltpu.async_copy(
                    t_any.at[pl.ds((c + _RING) * _PROJ_ROW, _PROJ_ROW)],
                    tbuf[s],
                    tsem[s],
                )

    for s in range(_RING):
        pltpu.make_async_copy(
            obuf[s], o_any.at[pl.ds(0, _PROJ_ROW)], osem[s]
        ).wait()


def _project_table(table, W, b):
    return pl.pallas_call(
        _proj_body,
        in_specs=[
            pl.BlockSpec(memory_space=pltpu.HBM),
            pl.BlockSpec(memory_space=pltpu.VMEM),
            pl.BlockSpec(memory_space=pltpu.VMEM),
        ],
        out_specs=pl.BlockSpec(memory_space=pltpu.HBM),
        out_shape=jax.ShapeDtypeStruct((_VOCAB, _OUT), jnp.float32),
        scratch_shapes=(
            [pltpu.VMEM((_PROJ_ROW, _EMB), jnp.float32)] * _RING
            + [pltpu.VMEM((_PROJ_ROW, _OUT), jnp.float32)] * _RING
            + [pltpu.SemaphoreType.DMA] * (2 * _RING)
        ),
    )(table, W, b.reshape(1, _OUT))


_NTOK = 4096 * 200
_NW = 32
_PER_W = _NTOK // _NW
_CHUNK = 400
_NCHUNK = _PER_W // _CHUNK


def _gather_kernel(tok_hbm, p_hbm, out_hbm, idx0, idx1, rows0, rows1, sem0, sem1):
    wid = lax.axis_index("s") * 2 + lax.axis_index("c")
    base = wid * _PER_W
    idx = (idx0, idx1)
    rows = (rows0, rows1)
    sem = (sem0, sem1)

    for b in range(2):
        pltpu.sync_copy(tok_hbm.at[pl.ds(base + b * _CHUNK, _CHUNK)], idx[b])
        pltpu.async_copy(p_hbm.at[idx[b]], rows[b], sem[b])

    @pl.loop(0, _NCHUNK - 2, step=2)
    def main(g):
        for b in range(2):
            c = g + b
            pltpu.make_async_copy(p_hbm.at[idx[b]], rows[b], sem[b]).wait()
            pltpu.sync_copy(rows[b], out_hbm.at[pl.ds(base + c * _CHUNK, _CHUNK)])
            pltpu.sync_copy(
                tok_hbm.at[pl.ds(base + (c + 2) * _CHUNK, _CHUNK)], idx[b]
            )
            pltpu.async_copy(p_hbm.at[idx[b]], rows[b], sem[b])

    for b in range(2):
        c = _NCHUNK - 2 + b
        pltpu.make_async_copy(p_hbm.at[idx[b]], rows[b], sem[b]).wait()
        pltpu.sync_copy(rows[b], out_hbm.at[pl.ds(base + c * _CHUNK, _CHUNK)])


@functools.partial(jax.jit, static_argnames=())
def kernel(tokens, table, W, b):
    proj = _project_table(table, W, b)
    tok_flat = tokens.reshape(_NTOK)

    sc_gather = pl.kernel(
        _gather_kernel,
        out_type=jax.ShapeDtypeStruct((_NTOK, _OUT), jnp.float32),
        mesh=plsc.VectorSubcoreMesh(core_axis_name="c", subcore_axis_name="s"),
        scratch_types=[
            pltpu.VMEM((_CHUNK,), jnp.int32),
            pltpu.VMEM((_CHUNK,), jnp.int32),
            pltpu.VMEM((_CHUNK, _OUT), jnp.float32),
            pltpu.VMEM((_CHUNK, _OUT), jnp.float32),
            pltpu.SemaphoreType.DMA,
            pltpu.SemaphoreType.DMA,
        ],
    )
    out_flat = sc_gather(tok_flat, proj)
    return out_flat.reshape(4096, 200, _OUT)
